# baseline (speedup 1.0000x reference)
import functools

import jax
import jax.numpy as jnp
from jax import lax
from jax.experimental import pallas as pl
from jax.experimental.pallas import tpu as pltpu
from jax.experimental.pallas import tpu_sc as plsc

B, N, S, DF = 16, 1024, 256, 256
NC, NS = 2, 16
NW = NC * NS
RPW = (B * N) // NW
CH = 64
NCH = RPW // CH
NBUF = 2


def _top3_body(x1_ref, x2_ref, out_ref):
    b = pl.program_id(0)
    x1t = x1_ref[0]
    x2 = x2_ref[0]
    d2 = jnp.zeros((S, N), jnp.float32)
    for c in range(3):
        diff = x2[:, c:c + 1] - x1t[c:c + 1, :]
        d2 = d2 + diff * diff
    dist = jnp.sqrt(jnp.maximum(d2, 0.0))
    iota = lax.broadcasted_iota(jnp.int32, (S, N), 0)
    base = b * S
    for k in range(3):
        m = jnp.min(dist, axis=0, keepdims=True)
        am = jnp.min(jnp.where(dist == m, iota, S), axis=0, keepdims=True)
        out_ref[0, pl.ds(k, 1), :] = (base + am) * 3 + k
        dist = jnp.where(iota == am, jnp.float32(jnp.inf), dist)


def _top3_indices(xyz1, xyz2):
    return pl.pallas_call(
        _top3_body,
        grid=(B,),
        in_specs=[
            pl.BlockSpec((1, 3, N), lambda b: (b, 0, 0)),
            pl.BlockSpec((1, S, 3), lambda b: (b, 0, 0)),
        ],
        out_specs=pl.BlockSpec((1, 3, N), lambda b: (b, 0, 0)),
        out_shape=jax.ShapeDtypeStruct((B, 3, N), jnp.int32),
    )(jnp.transpose(xyz1, (0, 2, 1)), xyz2)


def _sc_interp_body(table_hbm, fidx_hbm, out_hbm, idx_v, bufs,
                    gsem0, gsem1, ssem0, ssem1):
    gsems = (gsem0, gsem1)
    ssems = (ssem0, ssem1)
    wid = lax.axis_index("s") * NC + lax.axis_index("c")
    base = wid * RPW
    b_w = wid // 2
    noff = (wid % 2) * RPW
    for k in range(3):
        pltpu.sync_copy(
            fidx_hbm.at[pl.ds(b_w, 1), pl.ds(k, 1), pl.ds(noff, RPW)],
            idx_v.at[pl.ds(k, 1)])

    def gather(c, p):
        for k in range(3):
            pltpu.async_copy(
                table_hbm.at[idx_v.at[k, 0, pl.ds(c * CH, CH)]],
                bufs.at[p, k], gsems[p])

    def drain_gather(p):
        for k in range(3):
            pltpu.make_async_copy(
                table_hbm.at[pl.ds(0, CH)], bufs.at[p, k], gsems[p]).wait()

    def store(c, p):
        for k in range(3):
            pltpu.async_copy(
                bufs.at[p, k],
                out_hbm.at[k, pl.ds(base + c * CH, CH)], ssems[p])

    def drain_store(p):
        for k in range(3):
            pltpu.make_async_copy(
                bufs.at[p, k], out_hbm.at[0, pl.ds(0, CH)],
                ssems[p]).wait()

    for c in range(NBUF - 1):
        gather(c, c)

    def ring(g, carry):
        cbase = NBUF * g
        for p in range(NBUF):
            c = cbase + p
            pn = (p + NBUF - 1) % NBUF

            @pl.when(c + NBUF - 1 < NCH)
            def _():
                @pl.when(c - 1 >= 0)
                def _():
                    drain_store(pn)

                gather(c + NBUF - 1, pn)

            drain_gather(p)
            store(c, p)
        return carry

    lax.fori_loop(0, NCH // NBUF, ring, 0)
    for p in range(NBUF):
        drain_store(p)


@functools.cache
def _sc_interp_kernel():
    mesh = plsc.VectorSubcoreMesh(
        core_axis_name="c", subcore_axis_name="s", num_cores=NC)
    return pl.kernel(
        _sc_interp_body,
        mesh=mesh,
        out_type=jax.ShapeDtypeStruct((3, B * N, DF // 2), jnp.int32),
        scratch_types=[
            pltpu.VMEM((3, 1, RPW), jnp.int32),
            pltpu.VMEM((NBUF, 3, CH, DF // 2), jnp.int32),
            pltpu.SemaphoreType.DMA,
            pltpu.SemaphoreType.DMA,
            pltpu.SemaphoreType.DMA,
            pltpu.SemaphoreType.DMA,
        ],
    )


def _final_body(w0_ref, w1_ref, w2_ref, b0_ref, b1_ref, b2_ref,
                f1_ref, interp_ref, out_ref, weff_ref, weffb_bf_ref,
                beff_ref):
    @pl.when(pl.program_id(0) == 0)
    def _():
        w21 = jnp.dot(w2_ref[...], w1_ref[...],
                      preferred_element_type=jnp.float32)
        weff = jnp.dot(w21, w0_ref[...], preferred_element_type=jnp.float32)
        weff_ref[...] = weff[:, :S]
        weffb_bf_ref[...] = (weff[:, S:] * jnp.float32(1.0 / 3.0)).astype(
            jnp.bfloat16)
        dn = (((1,), (1,)), ((), ()))
        beff_row = (
            lax.dot_general(b0_ref[...], w21, dn,
                            preferred_element_type=jnp.float32)
            + lax.dot_general(b1_ref[...], w2_ref[...], dn,
                              preferred_element_type=jnp.float32)
            + b2_ref[...])
        beff_ref[...] = beff_row.T

    isum = interp_ref[0, 0] + interp_ref[1, 0] + interp_ref[2, 0]
    out_ref[0] = (
        jnp.dot(weff_ref[...], f1_ref[0], preferred_element_type=jnp.float32)
        + jnp.dot(weffb_bf_ref[...], isum,
                  preferred_element_type=jnp.float32)
        + beff_ref[...])


def _final_matmul(W0, W1, W2, b0, b1, b2, features1, interp):
    return pl.pallas_call(
        _final_body,
        grid=(B,),
        in_specs=[
            pl.BlockSpec((512, 1280), lambda b: (0, 0)),
            pl.BlockSpec((512, 512), lambda b: (0, 0)),
            pl.BlockSpec((256, 512), lambda b: (0, 0)),
            pl.BlockSpec((1, 512), lambda b: (0, 0)),
            pl.BlockSpec((1, 512), lambda b: (0, 0)),
            pl.BlockSpec((1, 256), lambda b: (0, 0)),
            pl.BlockSpec((1, S, DF), lambda b: (b, 0, 0)),
            pl.BlockSpec((3, 1, N, DF), lambda b: (0, b, 0, 0)),
        ],
        out_specs=pl.BlockSpec((1, 256, 256), lambda b: (b, 0, 0)),
        out_shape=jax.ShapeDtypeStruct((B, 256, 256), jnp.float32),
        scratch_shapes=[
            pltpu.VMEM((256, S), jnp.float32),
            pltpu.VMEM((256, N), jnp.bfloat16),
            pltpu.VMEM((256, 1), jnp.float32),
        ],
    )(W0, W1, W2, b0.reshape(1, 512), b1.reshape(1, 512), b2.reshape(1, 256),
      features1, interp.reshape(3, B, N, DF))


def kernel(xyz1, xyz2, features1, features2, W0, b0, W1, b1, W2, b2):
    fidx = _top3_indices(xyz1, xyz2)
    table = lax.bitcast_convert_type(
        features2.reshape(B * S * 3, DF // 2, 2).astype(jnp.bfloat16),
        jnp.int32)
    interp32 = _sc_interp_kernel()(table, fidx)
    interp = lax.bitcast_convert_type(interp32, jnp.bfloat16)
    return _final_matmul(W0, W1, W2, b0, b1, b2, features1, interp)


# speedup vs baseline: 30.1547x; 30.1547x over previous
import functools

import jax
import jax.numpy as jnp
from jax import lax
from jax.experimental import pallas as pl
from jax.experimental.pallas import tpu as pltpu
from jax.experimental.pallas import tpu_sc as plsc

B, N, S, DF = 16, 1024, 256, 256
NC, NS = 2, 16
NW = NC * NS
RPW = (B * N) // NW
CH = 64
NCH = RPW // CH
NBUF = 2


def _top3_body(x1_ref, x2_ref, out_ref):
    b = pl.program_id(0)
    x1t = x1_ref[0]
    x2 = x2_ref[0]
    d2 = jnp.zeros((S, N), jnp.float32)
    for c in range(3):
        diff = x2[:, c:c + 1] - x1t[c:c + 1, :]
        d2 = d2 + diff * diff
    dist = jnp.sqrt(jnp.maximum(d2, 0.0))
    iota = lax.broadcasted_iota(jnp.int32, (S, N), 0)
    base = b * S
    for k in range(3):
        m = jnp.min(dist, axis=0, keepdims=True)
        am = jnp.min(jnp.where(dist == m, iota, S), axis=0, keepdims=True)
        out_ref[0, pl.ds(k, 1), :] = (base + am) * 3 + k
        dist = jnp.where(iota == am, jnp.float32(jnp.inf), dist)


def _top3_indices(xyz1, xyz2):
    return pl.pallas_call(
        _top3_body,
        grid=(B,),
        in_specs=[
            pl.BlockSpec((1, 3, N), lambda b: (b, 0, 0)),
            pl.BlockSpec((1, S, 3), lambda b: (b, 0, 0)),
        ],
        out_specs=pl.BlockSpec((1, 3, N), lambda b: (b, 0, 0)),
        out_shape=jax.ShapeDtypeStruct((B, 3, N), jnp.int32),
    )(jnp.transpose(xyz1, (0, 2, 1)), xyz2)


def _sc_interp_body(table_hbm, fidx_hbm, out_hbm, idx_v, bufs,
                    gsem0, gsem1, ssem0, ssem1):
    gsems = (gsem0, gsem1)
    ssems = (ssem0, ssem1)
    wid = lax.axis_index("s") * NC + lax.axis_index("c")
    base = wid * RPW
    b_w = wid // 2
    noff = (wid % 2) * RPW
    for k in range(3):
        pltpu.sync_copy(
            fidx_hbm.at[pl.ds(b_w, 1), pl.ds(k, 1), pl.ds(noff, RPW)],
            idx_v.at[pl.ds(k, 1)])

    def gather(c, p):
        for k in range(3):
            pltpu.async_copy(
                table_hbm.at[idx_v.at[k, 0, pl.ds(c * CH, CH)]],
                bufs.at[p, k], gsems[p])

    def drain_gather(p):
        for k in range(3):
            pltpu.make_async_copy(
                table_hbm.at[pl.ds(0, CH)], bufs.at[p, k], gsems[p]).wait()

    def store(c, p):
        for k in range(3):
            pltpu.async_copy(
                bufs.at[p, k],
                out_hbm.at[k, pl.ds(base + c * CH, CH)], ssems[p])

    def drain_store(p):
        for k in range(3):
            pltpu.make_async_copy(
                bufs.at[p, k], out_hbm.at[0, pl.ds(0, CH)],
                ssems[p]).wait()

    for c in range(NBUF - 1):
        gather(c, c)

    def ring(g, carry):
        cbase = NBUF * g
        for p in range(NBUF):
            c = cbase + p
            pn = (p + NBUF - 1) % NBUF

            @pl.when(c + NBUF - 1 < NCH)
            def _():
                @pl.when(c - 1 >= 0)
                def _():
                    drain_store(pn)

                gather(c + NBUF - 1, pn)

            drain_gather(p)
            store(c, p)
        return carry

    lax.fori_loop(0, NCH // NBUF, ring, 0)
    for p in range(NBUF):
        drain_store(p)


@functools.cache
def _sc_interp_kernel():
    mesh = plsc.VectorSubcoreMesh(
        core_axis_name="c", subcore_axis_name="s", num_cores=NC)
    return pl.kernel(
        _sc_interp_body,
        mesh=mesh,
        out_type=jax.ShapeDtypeStruct((3, B * N, DF // 2), jnp.int32),
        scratch_types=[
            pltpu.VMEM((3, 1, RPW), jnp.int32),
            pltpu.VMEM((NBUF, 3, CH, DF // 2), jnp.int32),
            pltpu.SemaphoreType.DMA,
            pltpu.SemaphoreType.DMA,
            pltpu.SemaphoreType.DMA,
            pltpu.SemaphoreType.DMA,
        ],
    )


def _final_body(w0_ref, w1_ref, w2_ref, b0_ref, b1_ref, b2_ref,
                f1_ref, interp_ref, out_ref, weff_ref, weffb_bf_ref,
                beff_ref):
    @pl.when(pl.program_id(0) == 0)
    def _():
        w21 = jnp.dot(w2_ref[...], w1_ref[...],
                      preferred_element_type=jnp.float32)
        weff = jnp.dot(w21, w0_ref[...], preferred_element_type=jnp.float32)
        weff_ref[...] = weff[:, :S]
        weffb_bf_ref[...] = (weff[:, S:] * jnp.float32(1.0 / 3.0)).astype(
            jnp.bfloat16)
        dn = (((1,), (1,)), ((), ()))
        beff_row = (
            lax.dot_general(b0_ref[...], w21, dn,
                            preferred_element_type=jnp.float32)
            + lax.dot_general(b1_ref[...], w2_ref[...], dn,
                              preferred_element_type=jnp.float32)
            + b2_ref[...])
        beff_ref[...] = beff_row.T

    def lo(x):
        return lax.bitcast_convert_type(x << 16, jnp.float32)

    def hi(x):
        return lax.bitcast_convert_type(x & jnp.int32(-65536), jnp.float32)

    x0, x1, x2 = interp_ref[0, 0], interp_ref[1, 0], interp_ref[2, 0]
    isum_lo = (lo(x0) + lo(x1) + lo(x2)).astype(jnp.bfloat16)
    isum_hi = (hi(x0) + hi(x1) + hi(x2)).astype(jnp.bfloat16)
    base = (
        jnp.dot(weff_ref[...], f1_ref[0], preferred_element_type=jnp.float32)
        + beff_ref[...])
    out_ref[0, :, :DF // 2] = base[:, :DF // 2] + jnp.dot(
        weffb_bf_ref[...], isum_lo, preferred_element_type=jnp.float32)
    out_ref[0, :, DF // 2:] = base[:, DF // 2:] + jnp.dot(
        weffb_bf_ref[...], isum_hi, preferred_element_type=jnp.float32)


def _final_matmul(W0, W1, W2, b0, b1, b2, features1, interp):
    return pl.pallas_call(
        _final_body,
        grid=(B,),
        in_specs=[
            pl.BlockSpec((512, 1280), lambda b: (0, 0)),
            pl.BlockSpec((512, 512), lambda b: (0, 0)),
            pl.BlockSpec((256, 512), lambda b: (0, 0)),
            pl.BlockSpec((1, 512), lambda b: (0, 0)),
            pl.BlockSpec((1, 512), lambda b: (0, 0)),
            pl.BlockSpec((1, 256), lambda b: (0, 0)),
            pl.BlockSpec((1, S, DF), lambda b: (b, 0, 0)),
            pl.BlockSpec((3, 1, N, DF // 2), lambda b: (0, b, 0, 0)),
        ],
        out_specs=pl.BlockSpec((1, 256, 256), lambda b: (b, 0, 0)),
        out_shape=jax.ShapeDtypeStruct((B, 256, 256), jnp.float32),
        scratch_shapes=[
            pltpu.VMEM((256, S), jnp.float32),
            pltpu.VMEM((256, N), jnp.bfloat16),
            pltpu.VMEM((256, 1), jnp.float32),
        ],
    )(W0, W1, W2, b0.reshape(1, 512), b1.reshape(1, 512), b2.reshape(1, 256),
      features1, interp.reshape(3, B, N, DF // 2))


def kernel(xyz1, xyz2, features1, features2, W0, b0, W1, b1, W2, b2):
    fidx = _top3_indices(xyz1, xyz2)
    f2 = features2.reshape(B * S * 3, DF).astype(jnp.bfloat16)
    table = lax.bitcast_convert_type(
        jnp.stack([f2[:, :DF // 2], f2[:, DF // 2:]], axis=-1), jnp.int32)
    interp32 = _sc_interp_kernel()(table, fidx)
    return _final_matmul(W0, W1, W2, b0, b1, b2, features1, interp32)
